# Initial kernel scaffold; baseline (speedup 1.0000x reference)
import jax
import jax.numpy as jnp
from jax.experimental import pallas as pl


def kernel(feature, edge_index, W, b):
    raise NotImplementedError("write your pallas kernel here")


# speedup vs baseline: 2.9884x; 2.9884x over previous
import functools

import jax
import jax.numpy as jnp
from jax import lax
from jax.experimental import pallas as pl
from jax.experimental.pallas import tpu as pltpu
from jax.experimental.pallas import tpu_sc as plsc

N_NODES = 10000
N_EDGES = 320000
D = 128

NC = 2
NS = 16
NW = NC * NS
CH = 128
NCH = 80
EPW = CH * NCH
E_PAD = NW * EPW
N_ROWS = 10240
RPT = N_ROWS // NS
DUMMY = N_NODES


def _sc_body(feat_hbm, src_hbm, dst_hbm, out_hbm, srcv, dstv, buf, accum, sem):
    cid = lax.axis_index("c")
    sid = lax.axis_index("s")
    wid = cid * NS + sid

    zero16 = jnp.zeros((16,), jnp.float32)

    def zrow(i, c):
        for j in range(D // 16):
            buf[i, pl.ds(16 * j, 16)] = zero16
        return c

    lax.fori_loop(0, CH, zrow, 0)
    for t in range(RPT // CH):
        pltpu.sync_copy(buf, accum.at[pl.ds(sid * RPT + t * CH, CH)])
    plsc.subcore_barrier()

    pltpu.sync_copy(src_hbm.at[pl.ds(wid * NCH, NCH)], srcv)
    pltpu.sync_copy(dst_hbm.at[pl.ds(wid * NCH, NCH)], dstv)

    def body(j, c):
        pltpu.async_copy(feat_hbm.at[srcv.at[j]], buf, sem).wait()
        pltpu.sync_copy(buf, accum.at[dstv.at[j]], add=True)
        return c

    lax.fori_loop(0, NCH, body, 0)
    plsc.subcore_barrier()

    for t in range(RPT // CH):
        r = sid * RPT + t * CH
        pltpu.sync_copy(accum.at[pl.ds(r, CH)], out_hbm.at[cid, pl.ds(r, CH)])


_sc_gcn = functools.partial(
    pl.kernel,
    mesh=plsc.VectorSubcoreMesh(core_axis_name="c", subcore_axis_name="s"),
    out_type=jax.ShapeDtypeStruct((NC, N_ROWS, D), jnp.float32),
    scratch_types=[
        pltpu.VMEM((NCH, CH), jnp.int32),
        pltpu.VMEM((NCH, CH), jnp.int32),
        pltpu.VMEM((CH, D), jnp.float32),
        pltpu.VMEM_SHARED((N_ROWS, D), jnp.float32),
        pltpu.SemaphoreType.DMA,
    ],
)(_sc_body)


def _tc_body(p_ref, w_ref, b_ref, o_ref):
    x = p_ref[0] + p_ref[1]
    o_ref[...] = (
        lax.dot_general(x, w_ref[...], (((1,), (1,)), ((), ())),
                        preferred_element_type=jnp.float32)
        + b_ref[...]
    )


def _tc_linear(partials, W, b2):
    blk = 400
    return pl.pallas_call(
        _tc_body,
        grid=(N_NODES // blk,),
        in_specs=[
            pl.BlockSpec((NC, blk, D), lambda i: (0, i, 0)),
            pl.BlockSpec((D, D), lambda i: (0, 0)),
            pl.BlockSpec((1, D), lambda i: (0, 0)),
        ],
        out_specs=pl.BlockSpec((blk, D), lambda i: (i, 0)),
        out_shape=jax.ShapeDtypeStruct((N_NODES, D), jnp.float32),
    )(partials, W, b2)


def kernel(feature, edge_index, W, b):
    ei = edge_index.astype(jnp.int32)
    pad = E_PAD - N_EDGES
    src2 = jnp.concatenate(
        [ei[0], jnp.zeros((pad,), jnp.int32)]).reshape(NW * NCH, CH)
    dst2 = jnp.concatenate(
        [ei[1], jnp.full((pad,), DUMMY, jnp.int32)]).reshape(NW * NCH, CH)
    partials = _sc_gcn(feature, src2, dst2)
    return _tc_linear(partials, W, b.reshape(1, D))


# speedup vs baseline: 4.7960x; 1.6049x over previous
import functools

import jax
import jax.numpy as jnp
from jax import lax
from jax.experimental import pallas as pl
from jax.experimental.pallas import tpu as pltpu
from jax.experimental.pallas import tpu_sc as plsc

N_NODES = 10000
N_EDGES = 320000
D = 128
DH = D // 2

NC = 2
NS = 16
CH = 128
NCH = 160
EPW = CH * NCH
E_PAD = NS * EPW
N_ROWS = 10240
RPT = N_ROWS // NS
DUMMY = N_NODES
NB = 4


def _sc_body(feat_hbm, src_hbm, dst_hbm, out_hbm, srcv, dstv,
             b0, b1, b2, b3, accum,
             g0, g1, g2, g3, s0, s1, s2, s3):
    bufs = (b0, b1, b2, b3)
    gsem = (g0, g1, g2, g3)
    ssem = (s0, s1, s2, s3)
    cid = lax.axis_index("c")
    sid = lax.axis_index("s")

    zero16 = jnp.zeros((16,), jnp.float32)

    def zrow(i, c):
        for j in range(DH // 16):
            b0[i, pl.ds(16 * j, 16)] = zero16
        return c

    lax.fori_loop(0, CH, zrow, 0)
    for t in range(RPT // CH):
        pltpu.sync_copy(b0, accum.at[pl.ds(sid * RPT + t * CH, CH)])

    pltpu.sync_copy(src_hbm.at[pl.ds(sid * NCH, NCH)], srcv)
    pltpu.sync_copy(dst_hbm.at[pl.ds(sid * NCH, NCH)], dstv)

    off = cid * N_NODES

    def orow(i, c):
        for j in range(CH // 16):
            sl = pl.ds(16 * j, 16)
            srcv[i, sl] = srcv[i, sl] + off
        return c

    lax.fori_loop(0, NCH, orow, 0)
    plsc.subcore_barrier()

    def group(g, c):
        jj = g * NB
        cps = [pltpu.async_copy(feat_hbm.at[srcv.at[jj + i]], bufs[i], gsem[i])
               for i in range(NB)]
        scs = []
        for i in range(NB):
            cps[i].wait()
            scs.append(pltpu.async_copy(bufs[i], accum.at[dstv.at[jj + i]],
                                        ssem[i], add=True))
        for s in scs:
            s.wait()
        return c

    lax.fori_loop(0, NCH // NB, group, 0)
    plsc.subcore_barrier()

    for t in range(RPT // CH):
        r = sid * RPT + t * CH
        pltpu.sync_copy(accum.at[pl.ds(r, CH)], out_hbm.at[cid, pl.ds(r, CH)])


_sc_gcn = functools.partial(
    pl.kernel,
    mesh=plsc.VectorSubcoreMesh(core_axis_name="c", subcore_axis_name="s"),
    compiler_params=pltpu.CompilerParams(use_tc_tiling_on_sc=False),
    out_type=jax.ShapeDtypeStruct((NC, N_ROWS, DH), jnp.float32),
    scratch_types=[
        pltpu.VMEM((NCH, CH), jnp.int32),
        pltpu.VMEM((NCH, CH), jnp.int32),
        pltpu.VMEM((CH, DH), jnp.float32),
        pltpu.VMEM((CH, DH), jnp.float32),
        pltpu.VMEM((CH, DH), jnp.float32),
        pltpu.VMEM((CH, DH), jnp.float32),
        pltpu.VMEM_SHARED((N_ROWS, DH), jnp.float32),
        pltpu.SemaphoreType.DMA,
        pltpu.SemaphoreType.DMA,
        pltpu.SemaphoreType.DMA,
        pltpu.SemaphoreType.DMA,
        pltpu.SemaphoreType.DMA,
        pltpu.SemaphoreType.DMA,
        pltpu.SemaphoreType.DMA,
        pltpu.SemaphoreType.DMA,
    ],
)(_sc_body)


def _tc_body(p_ref, w_ref, b_ref, o_ref):
    x = jnp.concatenate([p_ref[0], p_ref[1]], axis=1)
    o_ref[...] = (
        lax.dot_general(x, w_ref[...], (((1,), (1,)), ((), ())),
                        preferred_element_type=jnp.float32)
        + b_ref[...]
    )


def _tc_linear(partials, W, b2):
    blk = 400
    return pl.pallas_call(
        _tc_body,
        grid=(N_NODES // blk,),
        in_specs=[
            pl.BlockSpec((NC, blk, DH), lambda i: (0, i, 0)),
            pl.BlockSpec((D, D), lambda i: (0, 0)),
            pl.BlockSpec((1, D), lambda i: (0, 0)),
        ],
        out_specs=pl.BlockSpec((blk, D), lambda i: (i, 0)),
        out_shape=jax.ShapeDtypeStruct((N_NODES, D), jnp.float32),
    )(partials, W, b2)


def kernel(feature, edge_index, W, b):
    ei = edge_index.astype(jnp.int32)
    pad = E_PAD - N_EDGES
    src2 = jnp.concatenate(
        [ei[0], jnp.zeros((pad,), jnp.int32)]).reshape(NS * NCH, CH)
    dst2 = jnp.concatenate(
        [ei[1], jnp.full((pad,), DUMMY, jnp.int32)]).reshape(NS * NCH, CH)
    fstk = jnp.concatenate([feature[:, :DH], feature[:, DH:]], axis=0)
    partials = _sc_gcn(fstk, src2, dst2)
    return _tc_linear(partials, W, b.reshape(1, D))
